# baseline (speedup 1.0000x reference)
import jax
import jax.numpy as jnp
from jax import lax
from jax.experimental import pallas as pl
from jax.experimental.pallas import tpu as pltpu
from jax.experimental.pallas import tpu_sc as plsc

_VOCAB = 1000000
_D = 64
_T = 128
_B = 4096
_S = 200

_NC = 2
_NS = 16
_NW = _NC * _NS
_BPW = _B // _NW
_V2 = 524288


def _tc_pair_table(emb_table):
    tt = emb_table.T
    q_blk = 4096
    n_blk = _V2 // q_blk

    def body(a_ref, b_ref, o_ref):
        o_ref[:, 0:_D] = a_ref[...].T
        o_ref[:, _D:2 * _D] = b_ref[...].T

    return pl.pallas_call(
        body,
        out_shape=jax.ShapeDtypeStruct((_V2, 2 * _D), jnp.float32),
        grid=(n_blk,),
        in_specs=[
            pl.BlockSpec((_D, q_blk), lambda i: (0, i)),
            pl.BlockSpec((_D, q_blk),
                         lambda i: (0, jnp.minimum(i + n_blk,
                                                   _VOCAB // q_blk))),
        ],
        out_specs=pl.BlockSpec((q_blk, 2 * _D), lambda i: (i, 0)),
        compiler_params=pltpu.CompilerParams(
            dimension_semantics=("parallel",)),
    )(tt, tt)


_NBUF = 4


def _sc_pool_body(xt_hbm, t2_hbm, out_hbm, idx_v, hv, sv, rv,
                  acc_sh, *sems):
    sid = lax.axis_index("s")
    wid = sid * _NC + lax.axis_index("c")
    base = wid * _BPW
    abase = sid * (2 * _BPW)

    @pl.loop(0, _BPW)
    def _(i):
        for k in range(2 * _D // 16):
            rv[0, i, pl.ds(16 * k, 16)] = jnp.zeros((16,), jnp.float32)

    pltpu.sync_copy(rv.at[0], acc_sh.at[pl.ds(abase, _BPW)])
    pltpu.sync_copy(rv.at[0], acc_sh.at[pl.ds(abase + _BPW, _BPW)])

    pltpu.sync_copy(xt_hbm.at[:, pl.ds(base, _BPW)], idx_v)

    def prep(j, b):
        for k in range(_BPW // 16):
            ids = idx_v[j, pl.ds(16 * k, 16)]
            big = ids >= _V2
            hv[b, pl.ds(16 * k, 16)] = ids - jnp.where(big, _V2, 0)
            sv[b, pl.ds(16 * k, 16)] = (lax.iota(jnp.int32, 16)
                                        + (16 * k + abase)
                                        + jnp.where(big, _BPW, 0))

    for b in range(_NBUF):
        prep(b, b)
        pltpu.async_copy(t2_hbm.at[hv.at[b]], rv.at[b], sems[b])

    @pl.loop(_NBUF, _S, step=_NBUF)
    def _(j):
        for b in range(_NBUF):
            pltpu.make_async_copy(t2_hbm.at[hv.at[b]], rv.at[b],
                                  sems[b]).wait()
            pltpu.sync_copy(rv.at[b], acc_sh.at[sv.at[b]], add=True)
            prep(j + b, b)
            pltpu.async_copy(t2_hbm.at[hv.at[b]], rv.at[b], sems[b])

    for b in range(_NBUF):
        pltpu.make_async_copy(t2_hbm.at[hv.at[b]], rv.at[b], sems[b]).wait()
        pltpu.sync_copy(rv.at[b], acc_sh.at[sv.at[b]], add=True)

    pltpu.sync_copy(acc_sh.at[pl.ds(abase, _BPW)], rv.at[0])
    pltpu.sync_copy(acc_sh.at[pl.ds(abase + _BPW, _BPW)], rv.at[1])

    @pl.loop(0, _BPW)
    def _(i):
        for k in range(_D // 16):
            rv[2, i, pl.ds(16 * k, 16)] = (
                rv[0, i, pl.ds(16 * k, 16)]
                + rv[1, i, pl.ds(_D + 16 * k, 16)])

    pltpu.sync_copy(rv.at[2].at[:, pl.ds(0, _D)],
                    out_hbm.at[pl.ds(base, _BPW)])


def _sc_pool(xt, t2):
    mesh = plsc.VectorSubcoreMesh(core_axis_name="c", subcore_axis_name="s")
    return pl.kernel(
        _sc_pool_body,
        out_type=jax.ShapeDtypeStruct((_B, _D), jnp.float32),
        mesh=mesh,
        scratch_types=[
            pltpu.VMEM((_S, _BPW), jnp.int32),
            pltpu.VMEM((_NBUF, _BPW), jnp.int32),
            pltpu.VMEM((_NBUF, _BPW), jnp.int32),
            pltpu.VMEM((_NBUF, _BPW, 2 * _D), jnp.float32),
            pltpu.VMEM_SHARED((_NS * 2 * _BPW, 2 * _D), jnp.float32),
        ] + [pltpu.SemaphoreType.DMA] * _NBUF,
        compiler_params=pltpu.CompilerParams(
            use_tc_tiling_on_sc=False, needs_layout_passes=False),
    )(xt, t2)


def _mm_body(p_ref, w_ref, b_ref, o_ref):
    o_ref[...] = (
        jnp.dot(p_ref[...], w_ref[...],
                preferred_element_type=jnp.float32,
                precision=lax.Precision.HIGHEST)
        + b_ref[...]
    )


def _tc_project(pooled, W, b):
    blk = 512
    return pl.pallas_call(
        _mm_body,
        out_shape=jax.ShapeDtypeStruct((_B, _T), jnp.float32),
        grid=(_B // blk,),
        in_specs=[
            pl.BlockSpec((blk, _D), lambda i: (i, 0)),
            pl.BlockSpec((_D, _T), lambda i: (0, 0)),
            pl.BlockSpec((1, _T), lambda i: (0, 0)),
        ],
        out_specs=pl.BlockSpec((blk, _T), lambda i: (i, 0)),
    )(pooled, W, b.reshape(1, _T))


def kernel(x, emb_table, W, b):
    xt = x.T
    t2 = _tc_pair_table(emb_table)
    pooled = _sc_pool(xt, t2)
    return _tc_project(pooled, W, b)


# speedup vs baseline: 1.0737x; 1.0737x over previous
import jax
import jax.numpy as jnp
from jax import lax
from jax.experimental import pallas as pl
from jax.experimental.pallas import tpu as pltpu
from jax.experimental.pallas import tpu_sc as plsc

_VOCAB = 1000000
_D = 64
_T = 128
_B = 4096
_S = 200

_NC = 2
_NS = 16
_NW = _NC * _NS
_BPW = _B // _NW
_V2 = 524288


def _tc_pair_table(emb_table):
    tt = emb_table.T
    q_blk = 8192
    n_blk = _V2 // q_blk

    def body(a_ref, b_ref, o_ref):
        o_ref[:, 0:_D] = a_ref[...].T
        o_ref[:, _D:2 * _D] = b_ref[...].T

    return pl.pallas_call(
        body,
        out_shape=jax.ShapeDtypeStruct((_V2, 2 * _D), jnp.float32),
        grid=(n_blk,),
        in_specs=[
            pl.BlockSpec((_D, q_blk), lambda i: (0, i)),
            pl.BlockSpec((_D, q_blk),
                         lambda i: (0, jnp.minimum(i + n_blk,
                                                   _VOCAB // q_blk))),
        ],
        out_specs=pl.BlockSpec((q_blk, 2 * _D), lambda i: (i, 0)),
        compiler_params=pltpu.CompilerParams(
            dimension_semantics=("parallel",)),
    )(tt, tt)


_NBUF = 4


def _sc_pool_body(xt_hbm, t2_hbm, out_hbm, idx_v, hv, sv, rv,
                  acc_sh, *sems):
    sid = lax.axis_index("s")
    wid = sid * _NC + lax.axis_index("c")
    base = wid * _BPW
    abase = sid * (2 * _BPW)

    @pl.loop(0, _BPW)
    def _(i):
        for k in range(2 * _D // 16):
            rv[0, i, pl.ds(16 * k, 16)] = jnp.zeros((16,), jnp.float32)

    pltpu.sync_copy(rv.at[0], acc_sh.at[pl.ds(abase, _BPW)])
    pltpu.sync_copy(rv.at[0], acc_sh.at[pl.ds(abase + _BPW, _BPW)])

    pltpu.sync_copy(xt_hbm.at[:, pl.ds(base, _BPW)], idx_v)

    def prep(j, b):
        for k in range(_BPW // 16):
            ids = idx_v[j, pl.ds(16 * k, 16)]
            big = ids >= _V2
            hv[b, pl.ds(16 * k, 16)] = ids - jnp.where(big, _V2, 0)
            sv[b, pl.ds(16 * k, 16)] = (lax.iota(jnp.int32, 16)
                                        + (16 * k + abase)
                                        + jnp.where(big, _BPW, 0))

    for b in range(_NBUF):
        prep(b, b)
        pltpu.async_copy(t2_hbm.at[hv.at[b]], rv.at[b], sems[b])

    @pl.loop(_NBUF, _S, step=_NBUF)
    def _(j):
        for b in range(_NBUF):
            pltpu.make_async_copy(t2_hbm.at[hv.at[b]], rv.at[b],
                                  sems[b]).wait()
            pltpu.sync_copy(rv.at[b], acc_sh.at[sv.at[b]], add=True)
            prep(j + b, b)
            pltpu.async_copy(t2_hbm.at[hv.at[b]], rv.at[b], sems[b])

    for b in range(_NBUF):
        pltpu.make_async_copy(t2_hbm.at[hv.at[b]], rv.at[b], sems[b]).wait()
        pltpu.sync_copy(rv.at[b], acc_sh.at[sv.at[b]], add=True)

    pltpu.sync_copy(acc_sh.at[pl.ds(abase, _BPW)], rv.at[0])
    pltpu.sync_copy(acc_sh.at[pl.ds(abase + _BPW, _BPW)], rv.at[1])

    @pl.loop(0, _BPW)
    def _(i):
        for k in range(_D // 16):
            rv[2, i, pl.ds(16 * k, 16)] = (
                rv[0, i, pl.ds(16 * k, 16)]
                + rv[1, i, pl.ds(_D + 16 * k, 16)])

    pltpu.sync_copy(rv.at[2].at[:, pl.ds(0, _D)],
                    out_hbm.at[pl.ds(base, _BPW)])


def _sc_pool(xt, t2):
    mesh = plsc.VectorSubcoreMesh(core_axis_name="c", subcore_axis_name="s")
    return pl.kernel(
        _sc_pool_body,
        out_type=jax.ShapeDtypeStruct((_B, _D), jnp.float32),
        mesh=mesh,
        scratch_types=[
            pltpu.VMEM((_S, _BPW), jnp.int32),
            pltpu.VMEM((_NBUF, _BPW), jnp.int32),
            pltpu.VMEM((_NBUF, _BPW), jnp.int32),
            pltpu.VMEM((_NBUF, _BPW, 2 * _D), jnp.float32),
            pltpu.VMEM_SHARED((_NS * 2 * _BPW, 2 * _D), jnp.float32),
        ] + [pltpu.SemaphoreType.DMA] * _NBUF,
        compiler_params=pltpu.CompilerParams(
            use_tc_tiling_on_sc=False, needs_layout_passes=False),
    )(xt, t2)


def _mm_body(p_ref, w_ref, b_ref, o_ref):
    o_ref[...] = (
        jnp.dot(p_ref[...], w_ref[...],
                preferred_element_type=jnp.float32,
                precision=lax.Precision.HIGHEST)
        + b_ref[...]
    )


def _tc_project(pooled, W, b):
    blk = 512
    return pl.pallas_call(
        _mm_body,
        out_shape=jax.ShapeDtypeStruct((_B, _T), jnp.float32),
        grid=(_B // blk,),
        in_specs=[
            pl.BlockSpec((blk, _D), lambda i: (i, 0)),
            pl.BlockSpec((_D, _T), lambda i: (0, 0)),
            pl.BlockSpec((1, _T), lambda i: (0, 0)),
        ],
        out_specs=pl.BlockSpec((blk, _T), lambda i: (i, 0)),
    )(pooled, W, b.reshape(1, _T))


def kernel(x, emb_table, W, b):
    xt = x.T
    t2 = _tc_pair_table(emb_table)
    pooled = _sc_pool(xt, t2)
    return _tc_project(pooled, W, b)


# speedup vs baseline: 1.1106x; 1.0344x over previous
import jax
import jax.numpy as jnp
from jax import lax
from jax.experimental import pallas as pl
from jax.experimental.pallas import tpu as pltpu
from jax.experimental.pallas import tpu_sc as plsc

_VOCAB = 1000000
_D = 64
_T = 128
_B = 4096
_S = 200

_NC = 2
_NS = 16
_NW = _NC * _NS
_BPW = _B // _NW
_V2 = 524288


def _tc_pair_table(emb_table):
    tt = emb_table.T
    q_blk = 16384
    n_blk = _V2 // q_blk

    def body(a_ref, b_ref, o_ref):
        o_ref[:, 0:_D] = a_ref[...].T
        o_ref[:, _D:2 * _D] = b_ref[...].T

    return pl.pallas_call(
        body,
        out_shape=jax.ShapeDtypeStruct((_V2, 2 * _D), jnp.float32),
        grid=(n_blk,),
        in_specs=[
            pl.BlockSpec((_D, q_blk), lambda i: (0, i)),
            pl.BlockSpec((_D, q_blk),
                         lambda i: (0, jnp.minimum(i + n_blk,
                                                   _VOCAB // q_blk))),
        ],
        out_specs=pl.BlockSpec((q_blk, 2 * _D), lambda i: (i, 0)),
        compiler_params=pltpu.CompilerParams(
            dimension_semantics=("parallel",)),
    )(tt, tt)


_NBUF = 4


def _sc_pool_body(xt_hbm, t2_hbm, out_hbm, idx_v, hv, sv, rv,
                  acc_sh, *sems):
    sid = lax.axis_index("s")
    wid = sid * _NC + lax.axis_index("c")
    base = wid * _BPW
    abase = sid * (2 * _BPW)

    @pl.loop(0, _BPW)
    def _(i):
        for k in range(2 * _D // 16):
            rv[0, i, pl.ds(16 * k, 16)] = jnp.zeros((16,), jnp.float32)

    pltpu.sync_copy(rv.at[0], acc_sh.at[pl.ds(abase, _BPW)])
    pltpu.sync_copy(rv.at[0], acc_sh.at[pl.ds(abase + _BPW, _BPW)])

    pltpu.sync_copy(xt_hbm.at[:, pl.ds(base, _BPW)], idx_v)

    def prep(j, b):
        for k in range(_BPW // 16):
            ids = idx_v[j, pl.ds(16 * k, 16)]
            big = ids >= _V2
            hv[b, pl.ds(16 * k, 16)] = ids - jnp.where(big, _V2, 0)
            sv[b, pl.ds(16 * k, 16)] = (lax.iota(jnp.int32, 16)
                                        + (16 * k + abase)
                                        + jnp.where(big, _BPW, 0))

    for b in range(_NBUF):
        prep(b, b)
        pltpu.async_copy(t2_hbm.at[hv.at[b]], rv.at[b], sems[b])

    @pl.loop(_NBUF, _S, step=_NBUF)
    def _(j):
        for b in range(_NBUF):
            pltpu.make_async_copy(t2_hbm.at[hv.at[b]], rv.at[b],
                                  sems[b]).wait()
            pltpu.sync_copy(rv.at[b], acc_sh.at[sv.at[b]], add=True)
            prep(j + b, b)
            pltpu.async_copy(t2_hbm.at[hv.at[b]], rv.at[b], sems[b])

    for b in range(_NBUF):
        pltpu.make_async_copy(t2_hbm.at[hv.at[b]], rv.at[b], sems[b]).wait()
        pltpu.sync_copy(rv.at[b], acc_sh.at[sv.at[b]], add=True)

    pltpu.sync_copy(acc_sh.at[pl.ds(abase, _BPW)], rv.at[0])
    pltpu.sync_copy(acc_sh.at[pl.ds(abase + _BPW, _BPW)], rv.at[1])

    @pl.loop(0, _BPW)
    def _(i):
        for k in range(_D // 16):
            rv[2, i, pl.ds(16 * k, 16)] = (
                rv[0, i, pl.ds(16 * k, 16)]
                + rv[1, i, pl.ds(_D + 16 * k, 16)])

    pltpu.sync_copy(rv.at[2].at[:, pl.ds(0, _D)],
                    out_hbm.at[pl.ds(base, _BPW)])


def _sc_pool(xt, t2):
    mesh = plsc.VectorSubcoreMesh(core_axis_name="c", subcore_axis_name="s")
    return pl.kernel(
        _sc_pool_body,
        out_type=jax.ShapeDtypeStruct((_B, _D), jnp.float32),
        mesh=mesh,
        scratch_types=[
            pltpu.VMEM((_S, _BPW), jnp.int32),
            pltpu.VMEM((_NBUF, _BPW), jnp.int32),
            pltpu.VMEM((_NBUF, _BPW), jnp.int32),
            pltpu.VMEM((_NBUF, _BPW, 2 * _D), jnp.float32),
            pltpu.VMEM_SHARED((_NS * 2 * _BPW, 2 * _D), jnp.float32),
        ] + [pltpu.SemaphoreType.DMA] * _NBUF,
        compiler_params=pltpu.CompilerParams(
            use_tc_tiling_on_sc=False, needs_layout_passes=False),
    )(xt, t2)


def _mm_body(p_ref, w_ref, b_ref, o_ref):
    o_ref[...] = (
        jnp.dot(p_ref[...], w_ref[...],
                preferred_element_type=jnp.float32,
                precision=lax.Precision.HIGHEST)
        + b_ref[...]
    )


def _tc_project(pooled, W, b):
    blk = 512
    return pl.pallas_call(
        _mm_body,
        out_shape=jax.ShapeDtypeStruct((_B, _T), jnp.float32),
        grid=(_B // blk,),
        in_specs=[
            pl.BlockSpec((blk, _D), lambda i: (i, 0)),
            pl.BlockSpec((_D, _T), lambda i: (0, 0)),
            pl.BlockSpec((1, _T), lambda i: (0, 0)),
        ],
        out_specs=pl.BlockSpec((blk, _T), lambda i: (i, 0)),
    )(pooled, W, b.reshape(1, _T))


def kernel(x, emb_table, W, b):
    xt = x.T
    t2 = _tc_pair_table(emb_table)
    pooled = _sc_pool(xt, t2)
    return _tc_project(pooled, W, b)
